# baseline (speedup 1.0000x reference)
import functools

import jax
import jax.numpy as jnp
from jax import lax
from jax.experimental import pallas as pl
from jax.experimental.pallas import tpu as pltpu
from jax.experimental.pallas import tpu_sc as plsc

_B, _T, _H = 16, 2048, 1024
_HALF = _H // 2


def _gather_last_body(table_hbm, len_hbm, out_hbm, len_v, rows_v, sem):
    c = lax.axis_index("c")
    s = lax.axis_index("s")
    wid = s * 2 + c

    @pl.when(wid < 2)
    def _work():
        pltpu.sync_copy(len_hbm, len_v)
        lens = len_v[...]
        lane = lax.iota(jnp.int32, 16)
        t_fwd = jnp.maximum(jnp.minimum(lens - 1, _T - 1), 0)
        t = t_fwd * (1 - wid)
        row = 2 * (lane * _T + t) + wid
        pltpu.async_copy(table_hbm.at[row], rows_v, sem).wait()
        pltpu.sync_copy(rows_v, out_hbm.at[:, pl.ds(wid * _HALF, _HALF)])


def kernel(batch_hidden_states, video_fea, lengths):
    del video_fea
    table = batch_hidden_states.reshape(_B * _T * 2, _HALF)
    mesh = plsc.VectorSubcoreMesh(core_axis_name="c", subcore_axis_name="s")
    return pl.kernel(
        _gather_last_body,
        mesh=mesh,
        out_type=jax.ShapeDtypeStruct((_B, _H), jnp.float32),
        scratch_types=[
            pltpu.VMEM((_B,), jnp.int32),
            pltpu.VMEM((_B, _HALF), jnp.float32),
            pltpu.SemaphoreType.DMA,
        ],
    )(table, lengths)


# speedup vs baseline: 1.0172x; 1.0172x over previous
import functools

import jax
import jax.numpy as jnp
from jax import lax
from jax.experimental import pallas as pl
from jax.experimental.pallas import tpu as pltpu
from jax.experimental.pallas import tpu_sc as plsc

_B, _T, _H = 16, 2048, 1024
_HALF = _H // 2


def _gather_last_body(table_hbm, len_hbm, out_hbm, len_v, rows_v, sem):
    wid = lax.axis_index("s")

    @pl.when(wid < 2)
    def _work():
        pltpu.sync_copy(len_hbm, len_v)
        lens = len_v[...]
        lane = lax.iota(jnp.int32, 16)
        t_fwd = jnp.maximum(jnp.minimum(lens - 1, _T - 1), 0)
        t = t_fwd * (1 - wid)
        row = 2 * (lane * _T + t) + wid
        pltpu.async_copy(table_hbm.at[row], rows_v, sem).wait()
        pltpu.sync_copy(rows_v, out_hbm.at[:, pl.ds(wid * _HALF, _HALF)])


def kernel(batch_hidden_states, video_fea, lengths):
    del video_fea
    table = batch_hidden_states.reshape(_B * _T * 2, _HALF)
    mesh = plsc.VectorSubcoreMesh(
        core_axis_name="c", subcore_axis_name="s", num_cores=1
    )
    return pl.kernel(
        _gather_last_body,
        mesh=mesh,
        out_type=jax.ShapeDtypeStruct((_B, _H), jnp.float32),
        scratch_types=[
            pltpu.VMEM((_B,), jnp.int32),
            pltpu.VMEM((_B, _HALF), jnp.float32),
            pltpu.SemaphoreType.DMA,
        ],
    )(table, lengths)


# speedup vs baseline: 70.1882x; 69.0026x over previous
import jax
import jax.numpy as jnp
from jax.experimental import pallas as pl
from jax.experimental.pallas import tpu as pltpu

_B, _T, _H = 16, 2048, 1024
_HALF = _H // 2


def _gather_last_body(len_ref, bhs_ref, out_ref, sem):
    copies = [
        pltpu.make_async_copy(
            bhs_ref.at[:, 0, pl.ds(_HALF, _HALF)],
            out_ref.at[:, pl.ds(_HALF, _HALF)],
            sem,
        )
    ]
    for b in range(_B):
        t = jnp.clip(len_ref[b] - 1, 0, _T - 1)
        copies.append(
            pltpu.make_async_copy(
                bhs_ref.at[b, pl.ds(t, 1), pl.ds(0, _HALF)],
                out_ref.at[pl.ds(b, 1), pl.ds(0, _HALF)],
                sem,
            )
        )
    for c in copies:
        c.start()
    for c in copies:
        c.wait()


def kernel(batch_hidden_states, video_fea, lengths):
    del video_fea
    return pl.pallas_call(
        _gather_last_body,
        grid=(),
        in_specs=[
            pl.BlockSpec(memory_space=pltpu.MemorySpace.SMEM),
            pl.BlockSpec(memory_space=pl.ANY),
        ],
        out_specs=pl.BlockSpec(memory_space=pltpu.MemorySpace.VMEM),
        out_shape=jax.ShapeDtypeStruct((_B, _H), jnp.float32),
        scratch_shapes=[pltpu.SemaphoreType.DMA],
    )(lengths, batch_hidden_states)
